# Initial kernel scaffold; baseline (speedup 1.0000x reference)
import jax
import jax.numpy as jnp
from jax.experimental import pallas as pl


def kernel(x, edge_index, adj_values, W1, W2, u_gate_k, u_gate_b, u_cand_k, u_cand_b, i_gate_k, i_gate_b, i_cand_k, i_cand_b):
    raise NotImplementedError("write your pallas kernel here")


# speedup vs baseline: 2.9174x; 2.9174x over previous
import dataclasses
import functools

import jax
import jax.numpy as jnp
from jax import lax
from jax.experimental import pallas as pl
from jax.experimental.pallas import tpu as pltpu
from jax.experimental.pallas import tpu_sc as plsc

_N = 10000
_E = 320000
_D = 128
_HD = _D // 2
_USER = 6000

_NC = 2
_NS = 16
_NW = _NC * _NS
_CH = 80
_EPW = _E // _NW
_NCHUNK = _EPW // _CH
_ROWCH = 80

_BR = 1000


def _dot(a, b):
    return lax.dot_general(
        a, b, (((1,), (0,)), ((), ())),
        precision=lax.Precision.HIGHEST,
        preferred_element_type=jnp.float32)


def _spmm_partials(y_lo, y_hi, src3, dst3, vals3):
    mesh = plsc.VectorSubcoreMesh(core_axis_name="c", subcore_axis_name="s")
    cp = pltpu.CompilerParams(
        needs_layout_passes=False, use_tc_tiling_on_sc=False)

    @functools.partial(
        pl.kernel,
        out_type=jax.ShapeDtypeStruct((_NC, 2, _N, _HD), jnp.float32),
        mesh=mesh,
        compiler_params=cp,
        scratch_types=[
            pltpu.VMEM((_NCHUNK, _CH), jnp.int32),
            pltpu.VMEM((_NCHUNK, _CH), jnp.int32),
            pltpu.VMEM((_NCHUNK, _CH), jnp.float32),
            pltpu.VMEM((_CH, _HD), jnp.float32),
            pltpu.VMEM_SHARED((_N, _HD), jnp.float32),
        ],
    )
    def k(ylo_hbm, yhi_hbm, src_hbm, dst_hbm, vals_hbm, out_hbm,
          src_v, dst_v, vals_v, rows, acc):
        c = lax.axis_index("c")
        s = lax.axis_index("s")
        wid = s * _NC + c

        pltpu.sync_copy(src_hbm.at[wid], src_v)
        pltpu.sync_copy(dst_hbm.at[wid], dst_v)
        pltpu.sync_copy(vals_hbm.at[wid], vals_v)

        @pl.loop(0, _CH)
        def _(r):
            for q in range(_HD // 16):
                rows[r, pl.ds(16 * q, 16)] = jnp.zeros((16,), jnp.float32)

        for h, y_hbm in enumerate((ylo_hbm, yhi_hbm)):
            @pl.loop(s, _N // _ROWCH, step=_NS)
            def _(m):
                pltpu.sync_copy(rows, acc.at[pl.ds(m * _ROWCH, _ROWCH)])

            plsc.subcore_barrier()

            @pl.loop(0, _NCHUNK)
            def _(j):
                pltpu.sync_copy(y_hbm.at[src_v.at[j]], rows)
                jj = jnp.full((16,), j, dtype=jnp.int32)

                @pl.loop(0, _CH)
                def _(r):
                    vv = plsc.load_gather(
                        vals_v, [jj, jnp.full((16,), r, dtype=jnp.int32)])
                    for q in range(_HD // 16):
                        sl = pl.ds(16 * q, 16)
                        rows[r, sl] = rows[r, sl] * vv

                pltpu.sync_copy(rows, acc.at[dst_v.at[j]], add=True)

            plsc.subcore_barrier()

            @pl.loop(s, _N // _ROWCH, step=_NS)
            def _(m):
                sl = pl.ds(m * _ROWCH, _ROWCH)
                pltpu.sync_copy(acc.at[sl], out_hbm.at[c, h, sl])

            plsc.subcore_barrier()

            @pl.loop(0, _CH)
            def _(r):
                for q in range(_HD // 16):
                    rows[r, pl.ds(16 * q, 16)] = jnp.zeros((16,), jnp.float32)

    return k(y_lo, y_hi, src3, dst3, vals3)


def _halves(y):
    return y[:, :_HD], y[:, _HD:]


def _xw_kernel(x_ref, w_ref, lo_ref, hi_ref):
    lo_ref[...], hi_ref[...] = _halves(_dot(x_ref[...], w_ref[...]))


def _xw(x, w):
    return pl.pallas_call(
        _xw_kernel,
        grid=(_N // _BR,),
        in_specs=[pl.BlockSpec((_BR, _D), lambda i: (i, 0)),
                  pl.BlockSpec((_D, _D), lambda i: (0, 0))],
        out_specs=[pl.BlockSpec((_BR, _HD), lambda i: (i, 0)),
                   pl.BlockSpec((_BR, _HD), lambda i: (i, 0))],
        out_shape=[jax.ShapeDtypeStruct((_N, _HD), jnp.float32),
                   jax.ShapeDtypeStruct((_N, _HD), jnp.float32)],
    )(x, w)


def _merge_partials(p_ref):
    p = p_ref[...]
    return jnp.maximum(
        jnp.concatenate([p[0, 0] + p[1, 0], p[0, 1] + p[1, 1]], axis=-1), 0.0)


def _relu_mm_kernel(p_ref, w_ref, h_ref, lo_ref, hi_ref):
    h = _merge_partials(p_ref)
    h_ref[...] = h
    lo_ref[...], hi_ref[...] = _halves(_dot(h, w_ref[...]))


def _relu_mm(p, w):
    return pl.pallas_call(
        _relu_mm_kernel,
        grid=(_N // _BR,),
        in_specs=[pl.BlockSpec((_NC, 2, _BR, _HD), lambda i: (0, 0, i, 0)),
                  pl.BlockSpec((_D, _D), lambda i: (0, 0))],
        out_specs=[pl.BlockSpec((_BR, _D), lambda i: (i, 0)),
                   pl.BlockSpec((_BR, _HD), lambda i: (i, 0)),
                   pl.BlockSpec((_BR, _HD), lambda i: (i, 0))],
        out_shape=[jax.ShapeDtypeStruct((_N, _D), jnp.float32),
                   jax.ShapeDtypeStruct((_N, _HD), jnp.float32),
                   jax.ShapeDtypeStruct((_N, _HD), jnp.float32)],
    )(p, w)


def _gru_kernel(q_ref, x_ref, h1_ref,
                ugk_ref, ugb_ref, uck_ref, ucb_ref,
                igk_ref, igb_ref, ick_ref, icb_ref, o_ref):
    is_user = pl.program_id(0) < (_USER // _BR)
    gk = jnp.where(is_user, ugk_ref[...], igk_ref[...])
    gb = jnp.where(is_user, ugb_ref[...], igb_ref[...])
    ck = jnp.where(is_user, uck_ref[...], ick_ref[...])
    cb = jnp.where(is_user, ucb_ref[...], icb_ref[...])
    gk_x, gk_h = gk[:_D], gk[_D:]
    ck_x, ck_h = ck[:_D], ck[_D:]

    h2 = _merge_partials(q_ref)
    xt0 = x_ref[...]
    h = jnp.zeros_like(xt0)
    for xt in (xt0, h1_ref[...], h2):
        g = jax.nn.sigmoid(_dot(xt, gk_x) + _dot(h, gk_h) + gb)
        r, u = g[:, :_D], g[:, _D:]
        cand = jnp.tanh(_dot(xt, ck_x) + _dot(r * h, ck_h) + cb)
        h = u * h + (1.0 - u) * cand
    o_ref[...] = h


def _gru(q, x, h1, ugk, ugb, uck, ucb, igk, igb, ick, icb):
    full = lambda shape: pl.BlockSpec(shape, lambda i: tuple(0 for _ in shape))
    return pl.pallas_call(
        _gru_kernel,
        grid=(_N // _BR,),
        in_specs=[pl.BlockSpec((_NC, 2, _BR, _HD), lambda i: (0, 0, i, 0)),
                  pl.BlockSpec((_BR, _D), lambda i: (i, 0)),
                  pl.BlockSpec((_BR, _D), lambda i: (i, 0)),
                  full((2 * _D, 2 * _D)), full((1, 2 * _D)),
                  full((2 * _D, _D)), full((1, _D)),
                  full((2 * _D, 2 * _D)), full((1, 2 * _D)),
                  full((2 * _D, _D)), full((1, _D))],
        out_specs=pl.BlockSpec((_BR, _D), lambda i: (i, 0)),
        out_shape=jax.ShapeDtypeStruct((_N, _D), jnp.float32),
    )(q, x, h1, ugk, ugb, uck, ucb, igk, igb, ick, icb)


def kernel(x, edge_index, adj_values, W1, W2,
           u_gate_k, u_gate_b, u_cand_k, u_cand_b,
           i_gate_k, i_gate_b, i_cand_k, i_cand_b):
    src3 = edge_index[0].reshape(_NW, _NCHUNK, _CH)
    dst3 = edge_index[1].reshape(_NW, _NCHUNK, _CH)
    vals3 = adj_values.reshape(_NW, _NCHUNK, _CH)

    y1_lo, y1_hi = _xw(x, W1)
    p1 = _spmm_partials(y1_lo, y1_hi, src3, dst3, vals3)
    h1, y2_lo, y2_hi = _relu_mm(p1, W2)
    p2 = _spmm_partials(y2_lo, y2_hi, src3, dst3, vals3)
    return _gru(p2, x, h1,
                u_gate_k, u_gate_b.reshape(1, -1),
                u_cand_k, u_cand_b.reshape(1, -1),
                i_gate_k, i_gate_b.reshape(1, -1),
                i_cand_k, i_cand_b.reshape(1, -1))


# speedup vs baseline: 5.2762x; 1.8085x over previous
import dataclasses
import functools

import jax
import jax.numpy as jnp
from jax import lax
from jax.experimental import pallas as pl
from jax.experimental.pallas import tpu as pltpu
from jax.experimental.pallas import tpu_sc as plsc

_N = 10000
_E = 320000
_D = 128
_HD = _D // 2
_USER = 6000

_NC = 2
_NS = 16
_NW = _NC * _NS
_CH = 125
_EPW = _E // _NW
_NCHUNK = _EPW // _CH
_NBUF = 4
_ROWCH = 80

_BR = 1000


def _dot(a, b):
    return lax.dot_general(
        a, b, (((1,), (0,)), ((), ())),
        precision=lax.Precision.HIGHEST,
        preferred_element_type=jnp.float32)


def _spmm_partials(y_lo, y_hi, src3, dst3, vals3):
    mesh = plsc.VectorSubcoreMesh(core_axis_name="c", subcore_axis_name="s")
    cp = pltpu.CompilerParams(
        needs_layout_passes=False, use_tc_tiling_on_sc=False)

    @functools.partial(
        pl.kernel,
        out_type=jax.ShapeDtypeStruct((_NC, 2, _N, _HD), jnp.float32),
        mesh=mesh,
        compiler_params=cp,
        scratch_types=(
            [pltpu.VMEM((_NCHUNK, _CH), jnp.int32),
             pltpu.VMEM((_NCHUNK, _CH), jnp.int32),
             pltpu.VMEM((_NCHUNK, _CH), jnp.float32),
             pltpu.VMEM((_ROWCH, _HD), jnp.float32),
             pltpu.VMEM_SHARED((_N, _HD), jnp.float32)]
            + [pltpu.VMEM((_CH, _HD), jnp.float32)] * _NBUF
            + [pltpu.SemaphoreType.DMA] * (2 * _NBUF)
        ),
    )
    def k(ylo_hbm, yhi_hbm, src_hbm, dst_hbm, vals_hbm, out_hbm,
          src_v, dst_v, vals_v, zblk, acc, *ring):
        rows = ring[:_NBUF]
        sg = ring[_NBUF:2 * _NBUF]
        ss = ring[2 * _NBUF:3 * _NBUF]
        c = lax.axis_index("c")
        s = lax.axis_index("s")
        wid = s * _NC + c

        pltpu.sync_copy(src_hbm.at[wid], src_v)
        pltpu.sync_copy(dst_hbm.at[wid], dst_v)
        pltpu.sync_copy(vals_hbm.at[wid], vals_v)

        @pl.loop(0, _ROWCH)
        def _(r):
            for q in range(_HD // 16):
                zblk[r, pl.ds(16 * q, 16)] = jnp.zeros((16,), jnp.float32)

        def scale(rows_b, j):
            jj = jnp.full((16,), j, dtype=jnp.int32)

            @pl.loop(0, _CH, unroll=5)
            def _(r):
                vv = plsc.load_gather(
                    vals_v, [jj, jnp.full((16,), r, dtype=jnp.int32)])
                for q in range(_HD // 16):
                    sl = pl.ds(16 * q, 16)
                    rows_b[r, sl] = rows_b[r, sl] * vv

        for h, y_hbm in enumerate((ylo_hbm, yhi_hbm)):
            @pl.loop(s, _N // _ROWCH, step=_NS)
            def _(m):
                pltpu.sync_copy(zblk, acc.at[pl.ds(m * _ROWCH, _ROWCH)])

            plsc.subcore_barrier()

            for b in range(_NBUF):
                pltpu.async_copy(y_hbm.at[src_v.at[b]], rows[b], sg[b])

            @pl.loop(0, _NCHUNK // _NBUF)
            def _(t):
                for b in range(_NBUF):
                    j = _NBUF * t + b
                    pltpu.make_async_copy(
                        y_hbm.at[src_v.at[j]], rows[b], sg[b]).wait()
                    scale(rows[b], j)
                    pltpu.async_copy(
                        rows[b], acc.at[dst_v.at[j]], ss[b], add=True)
                    pb = (b + _NBUF - 1) % _NBUF
                    pj = j + _NBUF - 1

                    @pl.when(jnp.logical_and(j >= 1, pj < _NCHUNK))
                    def _():
                        pltpu.make_async_copy(
                            rows[pb], acc.at[dst_v.at[pj - _NBUF]],
                            ss[pb]).wait()
                        pltpu.async_copy(
                            y_hbm.at[src_v.at[pj]], rows[pb], sg[pb])

            for b in range(_NBUF):
                pltpu.make_async_copy(
                    rows[b], acc.at[dst_v.at[_NCHUNK - _NBUF + b]],
                    ss[b]).wait()

            plsc.subcore_barrier()

            @pl.loop(s, _N // _ROWCH, step=_NS)
            def _(m):
                sl = pl.ds(m * _ROWCH, _ROWCH)
                pltpu.sync_copy(acc.at[sl], out_hbm.at[c, h, sl])

            plsc.subcore_barrier()

    return k(y_lo, y_hi, src3, dst3, vals3)


def _halves(y):
    return y[:, :_HD], y[:, _HD:]


def _xw_kernel(x_ref, w_ref, lo_ref, hi_ref):
    lo_ref[...], hi_ref[...] = _halves(_dot(x_ref[...], w_ref[...]))


def _xw(x, w):
    return pl.pallas_call(
        _xw_kernel,
        grid=(_N // _BR,),
        in_specs=[pl.BlockSpec((_BR, _D), lambda i: (i, 0)),
                  pl.BlockSpec((_D, _D), lambda i: (0, 0))],
        out_specs=[pl.BlockSpec((_BR, _HD), lambda i: (i, 0)),
                   pl.BlockSpec((_BR, _HD), lambda i: (i, 0))],
        out_shape=[jax.ShapeDtypeStruct((_N, _HD), jnp.float32),
                   jax.ShapeDtypeStruct((_N, _HD), jnp.float32)],
    )(x, w)


def _merge_partials(p_ref):
    p = p_ref[...]
    return jnp.maximum(
        jnp.concatenate([p[0, 0] + p[1, 0], p[0, 1] + p[1, 1]], axis=-1), 0.0)


def _relu_mm_kernel(p_ref, w_ref, h_ref, lo_ref, hi_ref):
    h = _merge_partials(p_ref)
    h_ref[...] = h
    lo_ref[...], hi_ref[...] = _halves(_dot(h, w_ref[...]))


def _relu_mm(p, w):
    return pl.pallas_call(
        _relu_mm_kernel,
        grid=(_N // _BR,),
        in_specs=[pl.BlockSpec((_NC, 2, _BR, _HD), lambda i: (0, 0, i, 0)),
                  pl.BlockSpec((_D, _D), lambda i: (0, 0))],
        out_specs=[pl.BlockSpec((_BR, _D), lambda i: (i, 0)),
                   pl.BlockSpec((_BR, _HD), lambda i: (i, 0)),
                   pl.BlockSpec((_BR, _HD), lambda i: (i, 0))],
        out_shape=[jax.ShapeDtypeStruct((_N, _D), jnp.float32),
                   jax.ShapeDtypeStruct((_N, _HD), jnp.float32),
                   jax.ShapeDtypeStruct((_N, _HD), jnp.float32)],
    )(p, w)


def _gru_kernel(q_ref, x_ref, h1_ref,
                ugk_ref, ugb_ref, uck_ref, ucb_ref,
                igk_ref, igb_ref, ick_ref, icb_ref, o_ref):
    is_user = pl.program_id(0) < (_USER // _BR)
    gk = jnp.where(is_user, ugk_ref[...], igk_ref[...])
    gb = jnp.where(is_user, ugb_ref[...], igb_ref[...])
    ck = jnp.where(is_user, uck_ref[...], ick_ref[...])
    cb = jnp.where(is_user, ucb_ref[...], icb_ref[...])
    gk_x, gk_h = gk[:_D], gk[_D:]
    ck_x, ck_h = ck[:_D], ck[_D:]

    h2 = _merge_partials(q_ref)
    xt0 = x_ref[...]
    h = jnp.zeros_like(xt0)
    for xt in (xt0, h1_ref[...], h2):
        g = jax.nn.sigmoid(_dot(xt, gk_x) + _dot(h, gk_h) + gb)
        r, u = g[:, :_D], g[:, _D:]
        cand = jnp.tanh(_dot(xt, ck_x) + _dot(r * h, ck_h) + cb)
        h = u * h + (1.0 - u) * cand
    o_ref[...] = h


def _gru(q, x, h1, ugk, ugb, uck, ucb, igk, igb, ick, icb):
    full = lambda shape: pl.BlockSpec(shape, lambda i: tuple(0 for _ in shape))
    return pl.pallas_call(
        _gru_kernel,
        grid=(_N // _BR,),
        in_specs=[pl.BlockSpec((_NC, 2, _BR, _HD), lambda i: (0, 0, i, 0)),
                  pl.BlockSpec((_BR, _D), lambda i: (i, 0)),
                  pl.BlockSpec((_BR, _D), lambda i: (i, 0)),
                  full((2 * _D, 2 * _D)), full((1, 2 * _D)),
                  full((2 * _D, _D)), full((1, _D)),
                  full((2 * _D, 2 * _D)), full((1, 2 * _D)),
                  full((2 * _D, _D)), full((1, _D))],
        out_specs=pl.BlockSpec((_BR, _D), lambda i: (i, 0)),
        out_shape=jax.ShapeDtypeStruct((_N, _D), jnp.float32),
    )(q, x, h1, ugk, ugb, uck, ucb, igk, igb, ick, icb)


def kernel(x, edge_index, adj_values, W1, W2,
           u_gate_k, u_gate_b, u_cand_k, u_cand_b,
           i_gate_k, i_gate_b, i_cand_k, i_cand_b):
    src3 = edge_index[0].reshape(_NW, _NCHUNK, _CH)
    dst3 = edge_index[1].reshape(_NW, _NCHUNK, _CH)
    vals3 = adj_values.reshape(_NW, _NCHUNK, _CH)

    y1_lo, y1_hi = _xw(x, W1)
    p1 = _spmm_partials(y1_lo, y1_hi, src3, dst3, vals3)
    h1, y2_lo, y2_hi = _relu_mm(p1, W2)
    p2 = _spmm_partials(y2_lo, y2_hi, src3, dst3, vals3)
    return _gru(p2, x, h1,
                u_gate_k, u_gate_b.reshape(1, -1),
                u_cand_k, u_cand_b.reshape(1, -1),
                i_gate_k, i_gate_b.reshape(1, -1),
                i_cand_k, i_cand_b.reshape(1, -1))


# speedup vs baseline: 7.0672x; 1.3395x over previous
import dataclasses
import functools

import jax
import jax.numpy as jnp
from jax import lax
from jax.experimental import pallas as pl
from jax.experimental.pallas import tpu as pltpu
from jax.experimental.pallas import tpu_sc as plsc

_N = 10000
_E = 320000
_D = 128
_HD = _D // 2
_USER = 6000

_NC = 2
_NS = 16
_NW = _NC * _NS
_CH = 125
_EPW = _E // _NW
_NCHUNK = _EPW // _CH
_NBUF = 4
_ROWCH = 80

_BR = 1000


def _dot(a, b):
    return lax.dot_general(
        a, b, (((1,), (0,)), ((), ())),
        precision=lax.Precision.DEFAULT,
        preferred_element_type=jnp.float32)


def _spmm_partials(y_lo, y_hi, src3, dst3, vals3):
    mesh = plsc.VectorSubcoreMesh(core_axis_name="c", subcore_axis_name="s")
    cp = pltpu.CompilerParams(
        needs_layout_passes=False, use_tc_tiling_on_sc=False)

    @functools.partial(
        pl.kernel,
        out_type=jax.ShapeDtypeStruct((_NC, 2, _N, _HD), jnp.float32),
        mesh=mesh,
        compiler_params=cp,
        scratch_types=(
            [pltpu.VMEM((_NCHUNK, _CH), jnp.int32),
             pltpu.VMEM((_NCHUNK, _CH), jnp.int32),
             pltpu.VMEM((_NCHUNK, _CH), jnp.float32),
             pltpu.VMEM((_ROWCH, _HD), jnp.float32),
             pltpu.VMEM_SHARED((_N, _HD), jnp.float32)]
            + [pltpu.VMEM((_CH, _HD), jnp.float32)] * _NBUF
            + [pltpu.SemaphoreType.DMA] * (2 * _NBUF)
        ),
    )
    def k(ylo_hbm, yhi_hbm, src_hbm, dst_hbm, vals_hbm, out_hbm,
          src_v, dst_v, vals_v, zblk, acc, *ring):
        rows = ring[:_NBUF]
        sg = ring[_NBUF:2 * _NBUF]
        ss = ring[2 * _NBUF:3 * _NBUF]
        c = lax.axis_index("c")
        s = lax.axis_index("s")
        wid = s * _NC + c

        pltpu.sync_copy(src_hbm.at[wid], src_v)
        pltpu.sync_copy(dst_hbm.at[wid], dst_v)
        pltpu.sync_copy(vals_hbm.at[wid], vals_v)

        @pl.loop(0, _ROWCH)
        def _(r):
            for q in range(_HD // 16):
                zblk[r, pl.ds(16 * q, 16)] = jnp.zeros((16,), jnp.float32)

        def scale(rows_b, j):
            jj = jnp.full((16,), j, dtype=jnp.int32)

            @pl.loop(0, _CH, unroll=5)
            def _(r):
                vv = plsc.load_gather(
                    vals_v, [jj, jnp.full((16,), r, dtype=jnp.int32)])
                for q in range(_HD // 16):
                    sl = pl.ds(16 * q, 16)
                    rows_b[r, sl] = rows_b[r, sl] * vv

        for h, y_hbm in enumerate((ylo_hbm, yhi_hbm)):
            @pl.loop(s, _N // _ROWCH, step=_NS)
            def _(m):
                pltpu.sync_copy(zblk, acc.at[pl.ds(m * _ROWCH, _ROWCH)])

            plsc.subcore_barrier()

            for b in range(_NBUF):
                pltpu.async_copy(y_hbm.at[src_v.at[b]], rows[b], sg[b])

            @pl.loop(0, _NCHUNK // _NBUF)
            def _(t):
                for b in range(_NBUF):
                    j = _NBUF * t + b
                    pltpu.make_async_copy(
                        y_hbm.at[src_v.at[j]], rows[b], sg[b]).wait()
                    scale(rows[b], j)
                    pltpu.async_copy(
                        rows[b], acc.at[dst_v.at[j]], ss[b], add=True)
                    pb = (b + _NBUF - 1) % _NBUF
                    pj = j + _NBUF - 1

                    @pl.when(jnp.logical_and(j >= 1, pj < _NCHUNK))
                    def _():
                        pltpu.make_async_copy(
                            rows[pb], acc.at[dst_v.at[pj - _NBUF]],
                            ss[pb]).wait()
                        pltpu.async_copy(
                            y_hbm.at[src_v.at[pj]], rows[pb], sg[pb])

            for b in range(_NBUF):
                pltpu.make_async_copy(
                    rows[b], acc.at[dst_v.at[_NCHUNK - _NBUF + b]],
                    ss[b]).wait()

            plsc.subcore_barrier()

            @pl.loop(s, _N // _ROWCH, step=_NS)
            def _(m):
                sl = pl.ds(m * _ROWCH, _ROWCH)
                pltpu.sync_copy(acc.at[sl], out_hbm.at[c, h, sl])

            plsc.subcore_barrier()

    return k(y_lo, y_hi, src3, dst3, vals3)


def _halves(y):
    return y[:, :_HD], y[:, _HD:]


def _xw_kernel(x_ref, w_ref, lo_ref, hi_ref):
    lo_ref[...], hi_ref[...] = _halves(_dot(x_ref[...], w_ref[...]))


def _xw(x, w):
    return pl.pallas_call(
        _xw_kernel,
        grid=(_N // _BR,),
        in_specs=[pl.BlockSpec((_BR, _D), lambda i: (i, 0)),
                  pl.BlockSpec((_D, _D), lambda i: (0, 0))],
        out_specs=[pl.BlockSpec((_BR, _HD), lambda i: (i, 0)),
                   pl.BlockSpec((_BR, _HD), lambda i: (i, 0))],
        out_shape=[jax.ShapeDtypeStruct((_N, _HD), jnp.float32),
                   jax.ShapeDtypeStruct((_N, _HD), jnp.float32)],
    )(x, w)


def _merge_partials(p_ref):
    p = p_ref[...]
    return jnp.maximum(
        jnp.concatenate([p[0, 0] + p[1, 0], p[0, 1] + p[1, 1]], axis=-1), 0.0)


def _relu_mm_kernel(p_ref, w_ref, h_ref, lo_ref, hi_ref):
    h = _merge_partials(p_ref)
    h_ref[...] = h
    lo_ref[...], hi_ref[...] = _halves(_dot(h, w_ref[...]))


def _relu_mm(p, w):
    return pl.pallas_call(
        _relu_mm_kernel,
        grid=(_N // _BR,),
        in_specs=[pl.BlockSpec((_NC, 2, _BR, _HD), lambda i: (0, 0, i, 0)),
                  pl.BlockSpec((_D, _D), lambda i: (0, 0))],
        out_specs=[pl.BlockSpec((_BR, _D), lambda i: (i, 0)),
                   pl.BlockSpec((_BR, _HD), lambda i: (i, 0)),
                   pl.BlockSpec((_BR, _HD), lambda i: (i, 0))],
        out_shape=[jax.ShapeDtypeStruct((_N, _D), jnp.float32),
                   jax.ShapeDtypeStruct((_N, _HD), jnp.float32),
                   jax.ShapeDtypeStruct((_N, _HD), jnp.float32)],
    )(p, w)


def _gru_kernel(q_ref, x_ref, h1_ref,
                ugk_ref, ugb_ref, uck_ref, ucb_ref,
                igk_ref, igb_ref, ick_ref, icb_ref, o_ref):
    is_user = pl.program_id(0) < (_USER // _BR)
    gk = jnp.where(is_user, ugk_ref[...], igk_ref[...])
    gb = jnp.where(is_user, ugb_ref[...], igb_ref[...])
    ck = jnp.where(is_user, uck_ref[...], ick_ref[...])
    cb = jnp.where(is_user, ucb_ref[...], icb_ref[...])
    gk_x, gk_h = gk[:_D], gk[_D:]
    ck_x, ck_h = ck[:_D], ck[_D:]

    h2 = _merge_partials(q_ref)
    xt0 = x_ref[...]
    h = jnp.zeros_like(xt0)
    for xt in (xt0, h1_ref[...], h2):
        g = jax.nn.sigmoid(_dot(xt, gk_x) + _dot(h, gk_h) + gb)
        r, u = g[:, :_D], g[:, _D:]
        cand = jnp.tanh(_dot(xt, ck_x) + _dot(r * h, ck_h) + cb)
        h = u * h + (1.0 - u) * cand
    o_ref[...] = h


def _gru(q, x, h1, ugk, ugb, uck, ucb, igk, igb, ick, icb):
    full = lambda shape: pl.BlockSpec(shape, lambda i: tuple(0 for _ in shape))
    return pl.pallas_call(
        _gru_kernel,
        grid=(_N // _BR,),
        in_specs=[pl.BlockSpec((_NC, 2, _BR, _HD), lambda i: (0, 0, i, 0)),
                  pl.BlockSpec((_BR, _D), lambda i: (i, 0)),
                  pl.BlockSpec((_BR, _D), lambda i: (i, 0)),
                  full((2 * _D, 2 * _D)), full((1, 2 * _D)),
                  full((2 * _D, _D)), full((1, _D)),
                  full((2 * _D, 2 * _D)), full((1, 2 * _D)),
                  full((2 * _D, _D)), full((1, _D))],
        out_specs=pl.BlockSpec((_BR, _D), lambda i: (i, 0)),
        out_shape=jax.ShapeDtypeStruct((_N, _D), jnp.float32),
    )(q, x, h1, ugk, ugb, uck, ucb, igk, igb, ick, icb)


def kernel(x, edge_index, adj_values, W1, W2,
           u_gate_k, u_gate_b, u_cand_k, u_cand_b,
           i_gate_k, i_gate_b, i_cand_k, i_cand_b):
    src3 = edge_index[0].reshape(_NW, _NCHUNK, _CH)
    dst3 = edge_index[1].reshape(_NW, _NCHUNK, _CH)
    vals3 = adj_values.reshape(_NW, _NCHUNK, _CH)

    y1_lo, y1_hi = _xw(x, W1)
    p1 = _spmm_partials(y1_lo, y1_hi, src3, dst3, vals3)
    h1, y2_lo, y2_hi = _relu_mm(p1, W2)
    p2 = _spmm_partials(y2_lo, y2_hi, src3, dst3, vals3)
    return _gru(p2, x, h1,
                u_gate_k, u_gate_b.reshape(1, -1),
                u_cand_k, u_cand_b.reshape(1, -1),
                i_gate_k, i_gate_b.reshape(1, -1),
                i_cand_k, i_cand_b.reshape(1, -1))


# speedup vs baseline: 8.4997x; 1.2027x over previous
import dataclasses
import functools

import jax
import jax.numpy as jnp
from jax import lax
from jax.experimental import pallas as pl
from jax.experimental.pallas import tpu as pltpu
from jax.experimental.pallas import tpu_sc as plsc

_N = 10000
_E = 320000
_D = 128
_HD = _D // 2
_USER = 6000

_NC = 2
_NS = 16
_NW = _NC * _NS
_CH = 125
_EPW = _E // _NW
_NCHUNK = _EPW // _CH
_NBUF = 4
_ROWCH = 80

_BR = 1000


def _dot(a, b):
    return lax.dot_general(
        a, b, (((1,), (0,)), ((), ())),
        precision=lax.Precision.DEFAULT,
        preferred_element_type=jnp.float32)


def _spmm_partials(y_lo, y_hi, src3, dst3, vals3):
    mesh = plsc.VectorSubcoreMesh(core_axis_name="c", subcore_axis_name="s")
    cp = pltpu.CompilerParams(
        needs_layout_passes=False, use_tc_tiling_on_sc=False)

    @functools.partial(
        pl.kernel,
        out_type=jax.ShapeDtypeStruct((_NC, 2, _N, _HD), jnp.float32),
        mesh=mesh,
        compiler_params=cp,
        scratch_types=(
            [pltpu.VMEM((_NCHUNK, _CH), jnp.int32),
             pltpu.VMEM((_NCHUNK, _CH), jnp.int32),
             pltpu.VMEM((_NCHUNK, _CH), jnp.float32),
             pltpu.VMEM((_ROWCH, _HD), jnp.float32),
             pltpu.VMEM_SHARED((_N, _HD), jnp.float32)]
            + [pltpu.VMEM((_CH, _HD), jnp.float32)] * _NBUF
            + [pltpu.SemaphoreType.DMA] * (2 * _NBUF)
        ),
    )
    def k(ylo_hbm, yhi_hbm, src_hbm, dst_hbm, vals_hbm, out_hbm,
          src_v, dst_v, vals_v, zblk, acc, *ring):
        rows = ring[:_NBUF]
        sg = ring[_NBUF:2 * _NBUF]
        ss = ring[2 * _NBUF:3 * _NBUF]
        c = lax.axis_index("c")
        s = lax.axis_index("s")
        wid = s * _NC + c

        pltpu.sync_copy(src_hbm.at[wid], src_v)
        pltpu.sync_copy(dst_hbm.at[wid], dst_v)
        pltpu.sync_copy(vals_hbm.at[wid], vals_v)

        @pl.loop(0, _ROWCH)
        def _(r):
            for q in range(_HD // 16):
                zblk[r, pl.ds(16 * q, 16)] = jnp.zeros((16,), jnp.float32)

        def scale(rows_b, j):
            jj = jnp.full((16,), j, dtype=jnp.int32)

            @plsc.parallel_loop(0, _CH, unroll=5)
            def _(r):
                vv = plsc.load_gather(
                    vals_v, [jj, jnp.full((16,), r, dtype=jnp.int32)])
                for q in range(_HD // 16):
                    sl = pl.ds(16 * q, 16)
                    rows_b[r, sl] = rows_b[r, sl] * vv

        for h, y_hbm in enumerate((ylo_hbm, yhi_hbm)):
            @pl.loop(s, _N // _ROWCH, step=_NS)
            def _(m):
                pltpu.sync_copy(zblk, acc.at[pl.ds(m * _ROWCH, _ROWCH)])

            plsc.subcore_barrier()

            for b in range(_NBUF):
                pltpu.async_copy(y_hbm.at[src_v.at[b]], rows[b], sg[b])

            @pl.loop(0, _NCHUNK // _NBUF)
            def _(t):
                for b in range(_NBUF):
                    j = _NBUF * t + b
                    pltpu.make_async_copy(
                        y_hbm.at[src_v.at[j]], rows[b], sg[b]).wait()
                    scale(rows[b], j)
                    pltpu.async_copy(
                        rows[b], acc.at[dst_v.at[j]], ss[b], add=True)
                    pb = (b + _NBUF - 1) % _NBUF
                    pj = j + _NBUF - 1

                    @pl.when(jnp.logical_and(j >= 1, pj < _NCHUNK))
                    def _():
                        pltpu.make_async_copy(
                            rows[pb], acc.at[dst_v.at[pj - _NBUF]],
                            ss[pb]).wait()
                        pltpu.async_copy(
                            y_hbm.at[src_v.at[pj]], rows[pb], sg[pb])

            for b in range(_NBUF):
                pltpu.make_async_copy(
                    rows[b], acc.at[dst_v.at[_NCHUNK - _NBUF + b]],
                    ss[b]).wait()

            plsc.subcore_barrier()

            @pl.loop(s, _N // _ROWCH, step=_NS)
            def _(m):
                sl = pl.ds(m * _ROWCH, _ROWCH)
                pltpu.sync_copy(acc.at[sl], out_hbm.at[c, h, sl])

            plsc.subcore_barrier()

    return k(y_lo, y_hi, src3, dst3, vals3)


def _halves(y):
    return y[:, :_HD], y[:, _HD:]


def _xw_kernel(x_ref, w_ref, lo_ref, hi_ref):
    lo_ref[...], hi_ref[...] = _halves(_dot(x_ref[...], w_ref[...]))


def _xw(x, w):
    return pl.pallas_call(
        _xw_kernel,
        grid=(_N // _BR,),
        in_specs=[pl.BlockSpec((_BR, _D), lambda i: (i, 0)),
                  pl.BlockSpec((_D, _D), lambda i: (0, 0))],
        out_specs=[pl.BlockSpec((_BR, _HD), lambda i: (i, 0)),
                   pl.BlockSpec((_BR, _HD), lambda i: (i, 0))],
        out_shape=[jax.ShapeDtypeStruct((_N, _HD), jnp.float32),
                   jax.ShapeDtypeStruct((_N, _HD), jnp.float32)],
    )(x, w)


def _merge_partials(p_ref):
    p = p_ref[...]
    return jnp.maximum(
        jnp.concatenate([p[0, 0] + p[1, 0], p[0, 1] + p[1, 1]], axis=-1), 0.0)


def _relu_mm_kernel(p_ref, w_ref, h_ref, lo_ref, hi_ref):
    h = _merge_partials(p_ref)
    h_ref[...] = h
    lo_ref[...], hi_ref[...] = _halves(_dot(h, w_ref[...]))


def _relu_mm(p, w):
    return pl.pallas_call(
        _relu_mm_kernel,
        grid=(_N // _BR,),
        in_specs=[pl.BlockSpec((_NC, 2, _BR, _HD), lambda i: (0, 0, i, 0)),
                  pl.BlockSpec((_D, _D), lambda i: (0, 0))],
        out_specs=[pl.BlockSpec((_BR, _D), lambda i: (i, 0)),
                   pl.BlockSpec((_BR, _HD), lambda i: (i, 0)),
                   pl.BlockSpec((_BR, _HD), lambda i: (i, 0))],
        out_shape=[jax.ShapeDtypeStruct((_N, _D), jnp.float32),
                   jax.ShapeDtypeStruct((_N, _HD), jnp.float32),
                   jax.ShapeDtypeStruct((_N, _HD), jnp.float32)],
    )(p, w)


def _gru_kernel(q_ref, x_ref, h1_ref,
                ugk_ref, ugb_ref, uck_ref, ucb_ref,
                igk_ref, igb_ref, ick_ref, icb_ref, o_ref):
    is_user = pl.program_id(0) < (_USER // _BR)
    gk = jnp.where(is_user, ugk_ref[...], igk_ref[...])
    gb = jnp.where(is_user, ugb_ref[...], igb_ref[...])
    ck = jnp.where(is_user, uck_ref[...], ick_ref[...])
    cb = jnp.where(is_user, ucb_ref[...], icb_ref[...])
    gk_x, gk_h = gk[:_D], gk[_D:]
    ck_x, ck_h = ck[:_D], ck[_D:]

    h2 = _merge_partials(q_ref)
    xt0 = x_ref[...]
    h = jnp.zeros_like(xt0)
    for xt in (xt0, h1_ref[...], h2):
        g = jax.nn.sigmoid(_dot(xt, gk_x) + _dot(h, gk_h) + gb)
        r, u = g[:, :_D], g[:, _D:]
        cand = jnp.tanh(_dot(xt, ck_x) + _dot(r * h, ck_h) + cb)
        h = u * h + (1.0 - u) * cand
    o_ref[...] = h


def _gru(q, x, h1, ugk, ugb, uck, ucb, igk, igb, ick, icb):
    full = lambda shape: pl.BlockSpec(shape, lambda i: tuple(0 for _ in shape))
    return pl.pallas_call(
        _gru_kernel,
        grid=(_N // _BR,),
        in_specs=[pl.BlockSpec((_NC, 2, _BR, _HD), lambda i: (0, 0, i, 0)),
                  pl.BlockSpec((_BR, _D), lambda i: (i, 0)),
                  pl.BlockSpec((_BR, _D), lambda i: (i, 0)),
                  full((2 * _D, 2 * _D)), full((1, 2 * _D)),
                  full((2 * _D, _D)), full((1, _D)),
                  full((2 * _D, 2 * _D)), full((1, 2 * _D)),
                  full((2 * _D, _D)), full((1, _D))],
        out_specs=pl.BlockSpec((_BR, _D), lambda i: (i, 0)),
        out_shape=jax.ShapeDtypeStruct((_N, _D), jnp.float32),
    )(q, x, h1, ugk, ugb, uck, ucb, igk, igb, ick, icb)


def kernel(x, edge_index, adj_values, W1, W2,
           u_gate_k, u_gate_b, u_cand_k, u_cand_b,
           i_gate_k, i_gate_b, i_cand_k, i_cand_b):
    src3 = edge_index[0].reshape(_NW, _NCHUNK, _CH)
    dst3 = edge_index[1].reshape(_NW, _NCHUNK, _CH)
    vals3 = adj_values.reshape(_NW, _NCHUNK, _CH)

    y1_lo, y1_hi = _xw(x, W1)
    p1 = _spmm_partials(y1_lo, y1_hi, src3, dst3, vals3)
    h1, y2_lo, y2_hi = _relu_mm(p1, W2)
    p2 = _spmm_partials(y2_lo, y2_hi, src3, dst3, vals3)
    return _gru(p2, x, h1,
                u_gate_k, u_gate_b.reshape(1, -1),
                u_cand_k, u_cand_b.reshape(1, -1),
                i_gate_k, i_gate_b.reshape(1, -1),
                i_cand_k, i_cand_b.reshape(1, -1))


# speedup vs baseline: 8.8375x; 1.0397x over previous
import dataclasses
import functools

import jax
import jax.numpy as jnp
from jax import lax
from jax.experimental import pallas as pl
from jax.experimental.pallas import tpu as pltpu
from jax.experimental.pallas import tpu_sc as plsc

_N = 10000
_E = 320000
_D = 128
_HD = _D // 2
_USER = 6000

_NC = 2
_NS = 16
_NW = _NC * _NS
_CH = 125
_EPW = _E // _NW
_NCHUNK = _EPW // _CH
_NBUF = 4
_ROWCH = 80

_BR = 1000


def _dot(a, b):
    return lax.dot_general(
        a, b, (((1,), (0,)), ((), ())),
        precision=lax.Precision.DEFAULT,
        preferred_element_type=jnp.float32)


def _spmm_partials(y_lo, y_hi, src3, dst3, vals3):
    mesh = plsc.VectorSubcoreMesh(core_axis_name="c", subcore_axis_name="s")
    cp = pltpu.CompilerParams(
        needs_layout_passes=False, use_tc_tiling_on_sc=False)

    @functools.partial(
        pl.kernel,
        out_type=jax.ShapeDtypeStruct((_NC, 2, _N, _HD), jnp.float32),
        mesh=mesh,
        compiler_params=cp,
        scratch_types=(
            [pltpu.VMEM((_NCHUNK, _CH), jnp.int32),
             pltpu.VMEM((_NCHUNK, _CH), jnp.int32),
             pltpu.VMEM((_NCHUNK, _CH), jnp.float32),
             pltpu.VMEM((_ROWCH, _HD), jnp.float32),
             pltpu.VMEM_SHARED((_N, _HD), jnp.float32)]
            + [pltpu.VMEM((_CH, _HD), jnp.float32)] * _NBUF
            + [pltpu.SemaphoreType.DMA] * (2 * _NBUF)
        ),
    )
    def k(ylo_hbm, yhi_hbm, src_hbm, dst_hbm, vals_hbm, out_hbm,
          src_v, dst_v, vals_v, zblk, acc, *ring):
        rows = ring[:_NBUF]
        sg = ring[_NBUF:2 * _NBUF]
        ss = ring[2 * _NBUF:3 * _NBUF]
        c = lax.axis_index("c")
        s = lax.axis_index("s")
        wid = s * _NC + c

        pltpu.sync_copy(src_hbm.at[wid], src_v)
        pltpu.sync_copy(dst_hbm.at[wid], dst_v)
        pltpu.sync_copy(vals_hbm.at[wid], vals_v)

        @pl.loop(0, _ROWCH)
        def _(r):
            for q in range(_HD // 16):
                zblk[r, pl.ds(16 * q, 16)] = jnp.zeros((16,), jnp.float32)

        def scale(rows_b, j):
            jj = jnp.full((16,), j, dtype=jnp.int32)

            @plsc.parallel_loop(0, _CH, unroll=5)
            def _(r):
                vv = plsc.load_gather(
                    vals_v, [jj, jnp.full((16,), r, dtype=jnp.int32)])
                for q in range(_HD // 16):
                    sl = pl.ds(16 * q, 16)
                    rows_b[r, sl] = rows_b[r, sl] * vv

        for h, y_hbm in enumerate((ylo_hbm, yhi_hbm)):
            @pl.loop(s, _N // _ROWCH, step=_NS)
            def _(m):
                pltpu.sync_copy(zblk, acc.at[pl.ds(m * _ROWCH, _ROWCH)])

            plsc.subcore_barrier()

            for b in range(_NBUF):
                pltpu.async_copy(y_hbm.at[src_v.at[b]], rows[b], sg[b])

            @pl.loop(0, _NCHUNK // _NBUF)
            def _(t):
                for b in range(_NBUF):
                    j = _NBUF * t + b
                    pltpu.make_async_copy(
                        y_hbm.at[src_v.at[j]], rows[b], sg[b]).wait()
                    scale(rows[b], j)
                    pltpu.async_copy(
                        rows[b], acc.at[dst_v.at[j]], ss[b], add=True)
                    pb = (b + _NBUF - 1) % _NBUF
                    pj = j + _NBUF - 1

                    @pl.when(jnp.logical_and(j >= 1, pj < _NCHUNK))
                    def _():
                        pltpu.make_async_copy(
                            rows[pb], acc.at[dst_v.at[pj - _NBUF]],
                            ss[pb]).wait()
                        pltpu.async_copy(
                            y_hbm.at[src_v.at[pj]], rows[pb], sg[pb])

            for b in range(_NBUF):
                pltpu.make_async_copy(
                    rows[b], acc.at[dst_v.at[_NCHUNK - _NBUF + b]],
                    ss[b]).wait()

            plsc.subcore_barrier()

            @pl.loop(s, _N // _ROWCH, step=_NS)
            def _(m):
                sl = pl.ds(m * _ROWCH, _ROWCH)
                pltpu.sync_copy(acc.at[sl], out_hbm.at[c, h, sl])

            plsc.subcore_barrier()

    return k(y_lo, y_hi, src3, dst3, vals3)


def _halves(y):
    return y[:, :_HD], y[:, _HD:]


def _xw_kernel(x_ref, w_ref, lo_ref, hi_ref):
    lo_ref[...], hi_ref[...] = _halves(_dot(x_ref[...], w_ref[...]))


def _xw(x, w):
    return pl.pallas_call(
        _xw_kernel,
        grid=(_N // _BR,),
        in_specs=[pl.BlockSpec((_BR, _D), lambda i: (i, 0)),
                  pl.BlockSpec((_D, _D), lambda i: (0, 0))],
        out_specs=[pl.BlockSpec((_BR, _HD), lambda i: (i, 0)),
                   pl.BlockSpec((_BR, _HD), lambda i: (i, 0))],
        out_shape=[jax.ShapeDtypeStruct((_N, _HD), jnp.float32),
                   jax.ShapeDtypeStruct((_N, _HD), jnp.float32)],
    )(x, w)


def _merge_partials(p_ref):
    p = p_ref[...]
    return jnp.maximum(
        jnp.concatenate([p[0, 0] + p[1, 0], p[0, 1] + p[1, 1]], axis=-1), 0.0)


def _relu_mm_kernel(p_ref, w_ref, h_ref, lo_ref, hi_ref):
    h = _merge_partials(p_ref)
    h_ref[...] = h
    lo_ref[...], hi_ref[...] = _halves(_dot(h, w_ref[...]))


def _relu_mm(p, w):
    return pl.pallas_call(
        _relu_mm_kernel,
        grid=(_N // _BR,),
        in_specs=[pl.BlockSpec((_NC, 2, _BR, _HD), lambda i: (0, 0, i, 0)),
                  pl.BlockSpec((_D, _D), lambda i: (0, 0))],
        out_specs=[pl.BlockSpec((_BR, _D), lambda i: (i, 0)),
                   pl.BlockSpec((_BR, _HD), lambda i: (i, 0)),
                   pl.BlockSpec((_BR, _HD), lambda i: (i, 0))],
        out_shape=[jax.ShapeDtypeStruct((_N, _D), jnp.float32),
                   jax.ShapeDtypeStruct((_N, _HD), jnp.float32),
                   jax.ShapeDtypeStruct((_N, _HD), jnp.float32)],
    )(p, w)


def _pick_weights(wrefs):
    is_user = pl.program_id(0) < (_USER // _BR)
    ugk, ugb, uck, ucb, igk, igb, ick, icb = wrefs
    return (jnp.where(is_user, ugk[...], igk[...]),
            jnp.where(is_user, ugb[...], igb[...]),
            jnp.where(is_user, uck[...], ick[...]),
            jnp.where(is_user, ucb[...], icb[...]))


def _gru_step(xt, h, gk, gb, ck, cb):
    g = jax.nn.sigmoid(_dot(xt, gk[:_D]) + _dot(h, gk[_D:]) + gb)
    r, u = g[:, :_D], g[:, _D:]
    cand = jnp.tanh(_dot(xt, ck[:_D]) + _dot(r * h, ck[_D:]) + cb)
    return u * h + (1.0 - u) * cand


def _gru0_kernel(x_ref, *refs):
    gk, gb, ck, cb = _pick_weights(refs[:8])
    o_ref = refs[8]
    xt = x_ref[...]
    u = jax.nn.sigmoid(_dot(xt, gk[:_D]) + gb)[:, _D:]
    cand = jnp.tanh(_dot(xt, ck[:_D]) + cb)
    o_ref[...] = (1.0 - u) * cand


def _gru1_kernel(h1_ref, h0_ref, *refs):
    gk, gb, ck, cb = _pick_weights(refs[:8])
    refs[8][...] = _gru_step(h1_ref[...], h0_ref[...], gk, gb, ck, cb)


def _gru2_kernel(q_ref, ha_ref, *refs):
    gk, gb, ck, cb = _pick_weights(refs[:8])
    refs[8][...] = _gru_step(_merge_partials(q_ref), ha_ref[...],
                             gk, gb, ck, cb)


_FULL = lambda shape: pl.BlockSpec(shape, lambda i: tuple(0 for _ in shape))
_WSPECS = [_FULL((2 * _D, 2 * _D)), _FULL((1, 2 * _D)),
           _FULL((2 * _D, _D)), _FULL((1, _D)),
           _FULL((2 * _D, 2 * _D)), _FULL((1, 2 * _D)),
           _FULL((2 * _D, _D)), _FULL((1, _D))]
_ROWS = pl.BlockSpec((_BR, _D), lambda i: (i, 0))


def _gru_call(body, data_specs, *args):
    return pl.pallas_call(
        body,
        grid=(_N // _BR,),
        in_specs=list(data_specs) + _WSPECS,
        out_specs=_ROWS,
        out_shape=jax.ShapeDtypeStruct((_N, _D), jnp.float32),
    )(*args)


def kernel(x, edge_index, adj_values, W1, W2,
           u_gate_k, u_gate_b, u_cand_k, u_cand_b,
           i_gate_k, i_gate_b, i_cand_k, i_cand_b):
    src3 = edge_index[0].reshape(_NW, _NCHUNK, _CH)
    dst3 = edge_index[1].reshape(_NW, _NCHUNK, _CH)
    vals3 = adj_values.reshape(_NW, _NCHUNK, _CH)
    w = (u_gate_k, u_gate_b.reshape(1, -1),
         u_cand_k, u_cand_b.reshape(1, -1),
         i_gate_k, i_gate_b.reshape(1, -1),
         i_cand_k, i_cand_b.reshape(1, -1))

    y1_lo, y1_hi = _xw(x, W1)
    p1 = _spmm_partials(y1_lo, y1_hi, src3, dst3, vals3)
    h0 = _gru_call(_gru0_kernel, [_ROWS], x, *w)
    h1, y2_lo, y2_hi = _relu_mm(p1, W2)
    p2 = _spmm_partials(y2_lo, y2_hi, src3, dst3, vals3)
    ha = _gru_call(_gru1_kernel, [_ROWS, _ROWS], h1, h0, *w)
    qspec = pl.BlockSpec((_NC, 2, _BR, _HD), lambda i: (0, 0, i, 0))
    return _gru_call(_gru2_kernel, [qspec, _ROWS], p2, ha, *w)


# speedup vs baseline: 9.7879x; 1.1075x over previous
import dataclasses
import functools

import jax
import jax.numpy as jnp
from jax import lax
from jax.experimental import pallas as pl
from jax.experimental.pallas import tpu as pltpu
from jax.experimental.pallas import tpu_sc as plsc

_N = 10000
_E = 320000
_D = 128
_HD = _D // 2
_USER = 6000

_NC = 2
_NS = 16
_NW = _NC * _NS
_CH = 125
_EPW = _E // _NW
_NCHUNK = _EPW // _CH
_NBUF = 4
_ROWCH = 80

_BR = 1000


def _dot(a, b):
    return lax.dot_general(
        a, b, (((1,), (0,)), ((), ())),
        precision=lax.Precision.DEFAULT,
        preferred_element_type=jnp.float32)


def _spmm_partials(y_lo, y_hi, src3, dst3, vals3):
    mesh = plsc.VectorSubcoreMesh(core_axis_name="c", subcore_axis_name="s")
    cp = pltpu.CompilerParams(
        needs_layout_passes=False, use_tc_tiling_on_sc=False)

    @functools.partial(
        pl.kernel,
        out_type=jax.ShapeDtypeStruct((_NC, _N, _D), jnp.float32),
        mesh=mesh,
        compiler_params=cp,
        scratch_types=(
            [pltpu.VMEM((_NCHUNK, _CH), jnp.int32),
             pltpu.VMEM((_NCHUNK, _CH), jnp.int32),
             pltpu.VMEM((_NCHUNK, _CH), jnp.float32),
             pltpu.VMEM((_ROWCH, _HD), jnp.float32),
             pltpu.VMEM_SHARED((_N, _HD), jnp.float32)]
            + [pltpu.VMEM((_CH, _HD), jnp.float32)] * _NBUF
            + [pltpu.SemaphoreType.DMA] * (2 * _NBUF)
        ),
    )
    def k(ylo_hbm, yhi_hbm, src_hbm, dst_hbm, vals_hbm, out_hbm,
          src_v, dst_v, vals_v, zblk, acc, *ring):
        rows = ring[:_NBUF]
        sg = ring[_NBUF:2 * _NBUF]
        ss = ring[2 * _NBUF:3 * _NBUF]
        c = lax.axis_index("c")
        s = lax.axis_index("s")
        wid = s * _NC + c

        pltpu.sync_copy(src_hbm.at[wid], src_v)
        pltpu.sync_copy(dst_hbm.at[wid], dst_v)
        pltpu.sync_copy(vals_hbm.at[wid], vals_v)

        @pl.loop(0, _ROWCH)
        def _(r):
            for q in range(_HD // 16):
                zblk[r, pl.ds(16 * q, 16)] = jnp.zeros((16,), jnp.float32)

        def scale(rows_b, j):
            jj = jnp.full((16,), j, dtype=jnp.int32)

            @plsc.parallel_loop(0, _CH, unroll=5)
            def _(r):
                vv = plsc.load_gather(
                    vals_v, [jj, jnp.full((16,), r, dtype=jnp.int32)])
                for q in range(_HD // 16):
                    sl = pl.ds(16 * q, 16)
                    rows_b[r, sl] = rows_b[r, sl] * vv

        for h, y_hbm in enumerate((ylo_hbm, yhi_hbm)):
            @pl.loop(s, _N // _ROWCH, step=_NS)
            def _(m):
                pltpu.sync_copy(zblk, acc.at[pl.ds(m * _ROWCH, _ROWCH)])

            plsc.subcore_barrier()

            for b in range(_NBUF):
                pltpu.async_copy(y_hbm.at[src_v.at[b]], rows[b], sg[b])

            @pl.loop(0, _NCHUNK // _NBUF)
            def _(t):
                for b in range(_NBUF):
                    j = _NBUF * t + b
                    pltpu.make_async_copy(
                        y_hbm.at[src_v.at[j]], rows[b], sg[b]).wait()
                    scale(rows[b], j)
                    pltpu.async_copy(
                        rows[b], acc.at[dst_v.at[j]], ss[b], add=True)
                    pb = (b + _NBUF - 1) % _NBUF
                    pj = j + _NBUF - 1

                    @pl.when(jnp.logical_and(j >= 1, pj < _NCHUNK))
                    def _():
                        pltpu.make_async_copy(
                            rows[pb], acc.at[dst_v.at[pj - _NBUF]],
                            ss[pb]).wait()
                        pltpu.async_copy(
                            y_hbm.at[src_v.at[pj]], rows[pb], sg[pb])

            for b in range(_NBUF):
                pltpu.make_async_copy(
                    rows[b], acc.at[dst_v.at[_NCHUNK - _NBUF + b]],
                    ss[b]).wait()

            plsc.subcore_barrier()

            @pl.loop(s, _N // _ROWCH, step=_NS)
            def _(m):
                sl = pl.ds(m * _ROWCH, _ROWCH)
                pltpu.sync_copy(acc.at[sl],
                                out_hbm.at[c, sl, pl.ds(_HD * h, _HD)])

            plsc.subcore_barrier()

    return k(y_lo, y_hi, src3, dst3, vals3)


def _halves(y):
    return y[:, :_HD], y[:, _HD:]


def _xw_kernel(x_ref, w_ref, lo_ref, hi_ref):
    lo_ref[...], hi_ref[...] = _halves(_dot(x_ref[...], w_ref[...]))


def _xw(x, w):
    return pl.pallas_call(
        _xw_kernel,
        grid=(_N // _BR,),
        in_specs=[pl.BlockSpec((_BR, _D), lambda i: (i, 0)),
                  pl.BlockSpec((_D, _D), lambda i: (0, 0))],
        out_specs=[pl.BlockSpec((_BR, _HD), lambda i: (i, 0)),
                   pl.BlockSpec((_BR, _HD), lambda i: (i, 0))],
        out_shape=[jax.ShapeDtypeStruct((_N, _HD), jnp.float32),
                   jax.ShapeDtypeStruct((_N, _HD), jnp.float32)],
    )(x, w)


def _merge_partials(p_ref):
    p = p_ref[...]
    return jnp.maximum(p[0] + p[1], 0.0)


def _relu_mm_kernel(p_ref, w_ref, h_ref, lo_ref, hi_ref):
    h = _merge_partials(p_ref)
    h_ref[...] = h
    lo_ref[...], hi_ref[...] = _halves(_dot(h, w_ref[...]))


def _relu_mm(p, w):
    return pl.pallas_call(
        _relu_mm_kernel,
        grid=(_N // _BR,),
        in_specs=[pl.BlockSpec((_NC, _BR, _D), lambda i: (0, i, 0)),
                  pl.BlockSpec((_D, _D), lambda i: (0, 0))],
        out_specs=[pl.BlockSpec((_BR, _D), lambda i: (i, 0)),
                   pl.BlockSpec((_BR, _HD), lambda i: (i, 0)),
                   pl.BlockSpec((_BR, _HD), lambda i: (i, 0))],
        out_shape=[jax.ShapeDtypeStruct((_N, _D), jnp.float32),
                   jax.ShapeDtypeStruct((_N, _HD), jnp.float32),
                   jax.ShapeDtypeStruct((_N, _HD), jnp.float32)],
    )(p, w)


def _pick_weights(wrefs):
    is_user = pl.program_id(0) < (_USER // _BR)
    ugk, ugb, uck, ucb, igk, igb, ick, icb = wrefs
    return (jnp.where(is_user, ugk[...], igk[...]),
            jnp.where(is_user, ugb[...], igb[...]),
            jnp.where(is_user, uck[...], ick[...]),
            jnp.where(is_user, ucb[...], icb[...]))


def _gru_step(xt, h, gk, gb, ck, cb):
    g = jax.nn.sigmoid(_dot(xt, gk[:_D]) + _dot(h, gk[_D:]) + gb)
    r, u = g[:, :_D], g[:, _D:]
    cand = jnp.tanh(_dot(xt, ck[:_D]) + _dot(r * h, ck[_D:]) + cb)
    return u * h + (1.0 - u) * cand


def _gru0_kernel(x_ref, *refs):
    gk, gb, ck, cb = _pick_weights(refs[:8])
    o_ref = refs[8]
    xt = x_ref[...]
    u = jax.nn.sigmoid(_dot(xt, gk[:_D]) + gb)[:, _D:]
    cand = jnp.tanh(_dot(xt, ck[:_D]) + cb)
    o_ref[...] = (1.0 - u) * cand


def _gru1_kernel(h1_ref, h0_ref, *refs):
    gk, gb, ck, cb = _pick_weights(refs[:8])
    refs[8][...] = _gru_step(h1_ref[...], h0_ref[...], gk, gb, ck, cb)


def _gru2_kernel(q_ref, ha_ref, *refs):
    gk, gb, ck, cb = _pick_weights(refs[:8])
    refs[8][...] = _gru_step(_merge_partials(q_ref), ha_ref[...],
                             gk, gb, ck, cb)


_FULL = lambda shape: pl.BlockSpec(shape, lambda i: tuple(0 for _ in shape))
_WSPECS = [_FULL((2 * _D, 2 * _D)), _FULL((1, 2 * _D)),
           _FULL((2 * _D, _D)), _FULL((1, _D)),
           _FULL((2 * _D, 2 * _D)), _FULL((1, 2 * _D)),
           _FULL((2 * _D, _D)), _FULL((1, _D))]
_ROWS = pl.BlockSpec((_BR, _D), lambda i: (i, 0))


def _gru_call(body, data_specs, *args):
    return pl.pallas_call(
        body,
        grid=(_N // _BR,),
        in_specs=list(data_specs) + _WSPECS,
        out_specs=_ROWS,
        out_shape=jax.ShapeDtypeStruct((_N, _D), jnp.float32),
    )(*args)


def kernel(x, edge_index, adj_values, W1, W2,
           u_gate_k, u_gate_b, u_cand_k, u_cand_b,
           i_gate_k, i_gate_b, i_cand_k, i_cand_b):
    src3 = edge_index[0].reshape(_NW, _NCHUNK, _CH)
    dst3 = edge_index[1].reshape(_NW, _NCHUNK, _CH)
    vals3 = adj_values.reshape(_NW, _NCHUNK, _CH)
    w = (u_gate_k, u_gate_b.reshape(1, -1),
         u_cand_k, u_cand_b.reshape(1, -1),
         i_gate_k, i_gate_b.reshape(1, -1),
         i_cand_k, i_cand_b.reshape(1, -1))

    y1_lo, y1_hi = _xw(x, W1)
    p1 = _spmm_partials(y1_lo, y1_hi, src3, dst3, vals3)
    h0 = _gru_call(_gru0_kernel, [_ROWS], x, *w)
    h1, y2_lo, y2_hi = _relu_mm(p1, W2)
    p2 = _spmm_partials(y2_lo, y2_hi, src3, dst3, vals3)
    ha = _gru_call(_gru1_kernel, [_ROWS, _ROWS], h1, h0, *w)
    qspec = pl.BlockSpec((_NC, _BR, _D), lambda i: (0, i, 0))
    return _gru_call(_gru2_kernel, [qspec, _ROWS], p2, ha, *w)
